# baseline (speedup 1.0000x reference)
import jax
import jax.numpy as jnp
from jax import lax
from jax.experimental import pallas as pl
from jax.experimental.pallas import tpu as pltpu
from jax.experimental.pallas import tpu_sc as plsc

B = 16384
F = 26
V = 100000
D = 32

NC = 2
NS = 16
NW = NC * NS
BPW = B // NW
CH = 128
NJ = BPW // CH
L = 16
NSTEP = F * NJ
O = 12544
OT = V - 7 * O
GP = O
TCW = 2560


def _body(tab_hbm, idx_hbm, out_hbm, idx_v, buf_v, acc_v, sem0, sem1):
    c = lax.axis_index("c")
    s = lax.axis_index("s")
    wid = s * NC + c

    pltpu.sync_copy(idx_hbm.at[wid], idx_v)

    def prep(i, _):
        f = i // (BPW // L)
        r = i - f * (BPW // L)
        j = r // (CH // L)
        cc = r - j * (CH // L)
        sl = (f, j, pl.ds(cc * L, L))
        v = idx_v[sl]
        q = jnp.minimum(v // O, 7)
        idx_v[sl] = (f * GP + v - q * O) * 8 + q
        return 0
    lax.fori_loop(0, F * BPW // L, prep, 0)

    def zstep(i, _):
        acc_v[pl.ds(i * L, L)] = jnp.zeros((L,), jnp.float32)
        return 0
    lax.fori_loop(0, BPW * D // L, zstep, 0)

    sems = (sem0, sem1)

    def fire(g, slot):
        f = g // NJ
        j = g - f * NJ
        pltpu.async_copy(tab_hbm.at[idx_v.at[f, j]], buf_v.at[slot],
                         sems[slot])

    def drain(slot):
        pltpu.make_async_copy(tab_hbm.at[idx_v.at[0, 0]], buf_v.at[slot],
                              sems[slot]).wait()

    iota2 = lax.iota(jnp.int32, L) * 2
    mhi = jnp.full((L,), -65536, jnp.int32)

    def accumulate(g, slot):
        f = g // NJ
        j = g - f * NJ
        jbase = j * CH * D

        @plsc.parallel_loop(0, CH, unroll=4)
        def _al(r):
            x = buf_v[slot, r, :]
            ev = plsc.bitcast(lax.shift_left(x, 16), jnp.float32)
            od = plsc.bitcast(lax.bitwise_and(x, mhi), jnp.float32)
            pos = jbase + r * D + iota2
            plsc.addupdate_scatter(acc_v, [pos], ev)
            plsc.addupdate_scatter(acc_v, [pos + 1], od)

    fire(0, 0)

    def step(i, _):
        g0 = 2 * i
        fire(g0 + 1, 1)
        drain(0)
        accumulate(g0, 0)

        @pl.when(g0 + 2 < NSTEP)
        def _():
            fire(g0 + 2, 0)
        drain(1)
        accumulate(g0 + 1, 1)
        return 0
    lax.fori_loop(0, NSTEP // 2, step, 0)

    pltpu.sync_copy(acc_v, out_hbm.at[pl.ds(wid * BPW * D, BPW * D)])


def _tc_flatten_body(in_hbm, out_hbm, ibuf, obuf, sin0, sin1, so0, so1):
    sins = (sin0, sin1)
    souts = (so0, so1)
    K4 = GP // 4

    def fire_in(f, slot):
        for k in range(4):
            pltpu.async_copy(in_hbm.at[f, pl.ds(k * 8, 8)],
                             ibuf.at[slot, pl.ds(k * 8, 8)], sins[slot])

    def wait_in(slot):
        for k in range(4):
            pltpu.make_async_copy(in_hbm.at[0, pl.ds(k * 8, 8)],
                                  ibuf.at[slot, pl.ds(k * 8, 8)],
                                  sins[slot]).wait()

    def fire_out(f, slot):
        for k in range(4):
            pltpu.async_copy(obuf.at[slot, pl.ds(k * K4, K4)],
                             out_hbm.at[pl.ds(f * GP + k * K4, K4)],
                             souts[slot])

    def wait_out(slot):
        for k in range(4):
            pltpu.make_async_copy(obuf.at[slot, pl.ds(k * K4, K4)],
                                  out_hbm.at[pl.ds(k * K4, K4)],
                                  souts[slot]).wait()

    rr = lax.broadcasted_iota(jnp.int32, (D, D // 2), 0)
    cc2 = lax.broadcasted_iota(jnp.int32, (D, D // 2), 1)
    e_even = (rr == 2 * cc2).astype(jnp.float32)
    e_odd = (rr == 2 * cc2 + 1).astype(jnp.float32)

    def pack(p):
        ue = lax.bitcast_convert_type(jnp.dot(p, e_even), jnp.uint32)
        uo = lax.bitcast_convert_type(jnp.dot(p, e_odd), jnp.uint32)
        z = lax.bitwise_or(
            lax.shift_right_logical(ue, jnp.uint32(16)),
            lax.bitwise_and(uo, jnp.uint32(0xFFFF0000)))
        return lax.bitcast_convert_type(z, jnp.int32)

    def compute(slot):
        def win(h, _):
            r0 = h * TCW
            parts = [
                pack(ibuf[slot, :, pl.ds(s * O + r0, TCW)].T)
                for s in range(8)
            ]
            obuf[slot, pl.ds(r0, TCW), :] = jnp.concatenate(parts, axis=1)
            return 0
        nfull = GP // TCW
        lax.fori_loop(0, nfull, win, 0)
        r0 = nfull * TCW
        wt = GP - r0
        parts = [pack(ibuf[slot, :, s * O + r0:s * O + r0 + wt].T)
                 for s in range(7)]
        wr = OT - r0
        tail = pack(ibuf[slot, :, 7 * O + r0:7 * O + r0 + wr].T)
        parts.append(jnp.concatenate(
            [tail, jnp.zeros((wt - wr, D // 2), jnp.int32)], axis=0))
        obuf[slot, r0:r0 + wt, :] = jnp.concatenate(parts, axis=1)

    fire_in(0, 0)
    fire_in(1, 1)

    def step(i, _):
        for slot in range(2):
            f = 2 * i + slot
            wait_in(slot)

            @pl.when(i > 0)
            def _():
                wait_out(slot)
            compute(slot)
            fire_out(f, slot)

            @pl.when(f + 2 < F)
            def _():
                fire_in(f + 2, slot)
        return 0
    lax.fori_loop(0, F // 2, step, 0)
    wait_out(0)
    wait_out(1)


def _tc_flatten(tabt):
    return pl.pallas_call(
        _tc_flatten_body,
        grid=(1,),
        in_specs=[pl.BlockSpec(memory_space=pl.ANY)],
        out_specs=pl.BlockSpec(memory_space=pl.ANY),
        out_shape=jax.ShapeDtypeStruct((F * GP, 128), jnp.int32),
        scratch_shapes=[
            pltpu.VMEM((2, D, V), jnp.float32),
            pltpu.VMEM((2, GP, 128), jnp.int32),
            pltpu.SemaphoreType.DMA,
            pltpu.SemaphoreType.DMA,
            pltpu.SemaphoreType.DMA,
            pltpu.SemaphoreType.DMA,
        ],
        compiler_params=pltpu.CompilerParams(
            vmem_limit_bytes=60000 * 1024),
    )(tabt)


@jax.jit
def _encode(tab, idx4):
    mesh = plsc.VectorSubcoreMesh(
        core_axis_name="c", subcore_axis_name="s",
        num_cores=NC, num_subcores=NS)
    fn = pl.kernel(
        _body,
        out_type=jax.ShapeDtypeStruct((B * D,), jnp.float32),
        mesh=mesh,
        scratch_types=[
            pltpu.VMEM((F, NJ, CH), jnp.int32),
            pltpu.VMEM((2, CH, D // 2), jnp.int32),
            pltpu.VMEM((BPW * D,), jnp.float32),
            pltpu.SemaphoreType.DMA,
            pltpu.SemaphoreType.DMA,
        ],
        compiler_params=pltpu.CompilerParams(use_tc_tiling_on_sc=False,
                                             needs_layout_passes=False),
    )
    return fn(tab, idx4)


def kernel(x, tables):
    tab128 = _tc_flatten(tables.transpose(0, 2, 1))
    tab = tab128.reshape(F * GP * 8, D // 2)
    idx4 = x.reshape(NW, NJ, CH, F).transpose(0, 3, 1, 2)
    out_flat = _encode(tab, idx4)
    return out_flat.reshape(B, D)


# speedup vs baseline: 1.0643x; 1.0643x over previous
import jax
import jax.numpy as jnp
from jax import lax
from jax.experimental import pallas as pl
from jax.experimental.pallas import tpu as pltpu
from jax.experimental.pallas import tpu_sc as plsc

B = 16384
F = 26
V = 100000
D = 32

NC = 2
NS = 16
NW = NC * NS
BPW = B // NW
CH = 128
NJ = BPW // CH
L = 16
NSTEP = F * NJ
Q = 24960
GP = V - 3 * Q
TCW = 2560


def _body(tab_hbm, idx_hbm, out_hbm, idx_v, buf_v, acc_v, sem0, sem1):
    c = lax.axis_index("c")
    s = lax.axis_index("s")
    wid = s * NC + c

    pltpu.sync_copy(idx_hbm.at[wid], idx_v)

    def prep(i, _):
        f = i // (BPW // L)
        r = i - f * (BPW // L)
        j = r // (CH // L)
        cc = r - j * (CH // L)
        sl = (f, j, pl.ds(cc * L, L))
        v = idx_v[sl]
        q = jnp.minimum(v // Q, 3)
        idx_v[sl] = (f * GP + v - q * Q) * 4 + q
        return 0
    lax.fori_loop(0, F * BPW // L, prep, 0)

    def zstep(i, _):
        acc_v[pl.ds(i * L, L)] = jnp.zeros((L,), jnp.float32)
        return 0
    lax.fori_loop(0, BPW * D // L, zstep, 0)

    sems = (sem0, sem1)

    def fire(g, slot):
        f = g // NJ
        j = g - f * NJ
        pltpu.async_copy(tab_hbm.at[idx_v.at[f, j]], buf_v.at[slot],
                         sems[slot])

    def drain(slot):
        pltpu.make_async_copy(tab_hbm.at[idx_v.at[0, 0]], buf_v.at[slot],
                              sems[slot]).wait()

    def accumulate(g, slot):
        f = g // NJ
        j = g - f * NJ
        jbase = j * CH * D

        @plsc.parallel_loop(0, CH, unroll=4)
        def _al(r):
            for h in range(D // L):
                plsc.addupdate(acc_v.at[pl.ds(jbase + r * D + h * L, L)],
                               buf_v[slot, r, pl.ds(h * L, L)])

    fire(0, 0)

    def step(i, _):
        g0 = 2 * i
        fire(g0 + 1, 1)
        drain(0)
        accumulate(g0, 0)

        @pl.when(g0 + 2 < NSTEP)
        def _():
            fire(g0 + 2, 0)
        drain(1)
        accumulate(g0 + 1, 1)
        return 0
    lax.fori_loop(0, NSTEP // 2, step, 0)

    pltpu.sync_copy(acc_v, out_hbm.at[pl.ds(wid * BPW * D, BPW * D)])


def _tc_flatten_body(in_hbm, out_hbm, ibuf, obuf, sin0, sin1, so0, so1):
    sins = (sin0, sin1)
    souts = (so0, so1)
    K4 = GP // 4

    def fire_in(f, slot):
        for k in range(4):
            pltpu.async_copy(in_hbm.at[f, pl.ds(k * 8, 8)],
                             ibuf.at[slot, pl.ds(k * 8, 8)], sins[slot])

    def wait_in(slot):
        for k in range(4):
            pltpu.make_async_copy(in_hbm.at[0, pl.ds(k * 8, 8)],
                                  ibuf.at[slot, pl.ds(k * 8, 8)],
                                  sins[slot]).wait()

    def fire_out(f, slot):
        for k in range(4):
            pltpu.async_copy(obuf.at[slot, pl.ds(k * K4, K4)],
                             out_hbm.at[pl.ds(f * GP + k * K4, K4)],
                             souts[slot])

    def wait_out(slot):
        for k in range(4):
            pltpu.make_async_copy(obuf.at[slot, pl.ds(k * K4, K4)],
                                  out_hbm.at[pl.ds(k * K4, K4)],
                                  souts[slot]).wait()

    def compute(slot):
        def win(h, _):
            r0 = h * TCW
            parts = [
                ibuf[slot, :, pl.ds(s * Q + r0, TCW)].T
                for s in range(4)
            ]
            obuf[slot, pl.ds(r0, TCW), :] = jnp.concatenate(parts, axis=1)
            return 0
        nfull = GP // TCW
        lax.fori_loop(0, nfull, win, 0)
        r0 = nfull * TCW
        wt = GP - r0
        parts = [ibuf[slot, :, s * Q + r0:s * Q + r0 + wt].T
                 for s in range(4)]
        obuf[slot, r0:r0 + wt, :] = jnp.concatenate(parts, axis=1)

    fire_in(0, 0)
    fire_in(1, 1)

    def step(i, _):
        for slot in range(2):
            f = 2 * i + slot
            wait_in(slot)

            @pl.when(i > 0)
            def _():
                wait_out(slot)
            compute(slot)
            fire_out(f, slot)

            @pl.when(f + 2 < F)
            def _():
                fire_in(f + 2, slot)
        return 0
    lax.fori_loop(0, F // 2, step, 0)
    wait_out(0)
    wait_out(1)


def _tc_flatten(tabt):
    return pl.pallas_call(
        _tc_flatten_body,
        grid=(1,),
        in_specs=[pl.BlockSpec(memory_space=pl.ANY)],
        out_specs=pl.BlockSpec(memory_space=pl.ANY),
        out_shape=jax.ShapeDtypeStruct((F * GP, 128), jnp.float32),
        scratch_shapes=[
            pltpu.VMEM((2, D, V), jnp.float32),
            pltpu.VMEM((2, GP, 128), jnp.float32),
            pltpu.SemaphoreType.DMA,
            pltpu.SemaphoreType.DMA,
            pltpu.SemaphoreType.DMA,
            pltpu.SemaphoreType.DMA,
        ],
        compiler_params=pltpu.CompilerParams(
            vmem_limit_bytes=60000 * 1024),
    )(tabt)


@jax.jit
def _encode(tab, idx4):
    mesh = plsc.VectorSubcoreMesh(
        core_axis_name="c", subcore_axis_name="s",
        num_cores=NC, num_subcores=NS)
    fn = pl.kernel(
        _body,
        out_type=jax.ShapeDtypeStruct((B * D,), jnp.float32),
        mesh=mesh,
        scratch_types=[
            pltpu.VMEM((F, NJ, CH), jnp.int32),
            pltpu.VMEM((2, CH, D), jnp.float32),
            pltpu.VMEM((BPW * D,), jnp.float32),
            pltpu.SemaphoreType.DMA,
            pltpu.SemaphoreType.DMA,
        ],
        compiler_params=pltpu.CompilerParams(use_tc_tiling_on_sc=False,
                                             needs_layout_passes=False),
    )
    return fn(tab, idx4)


def kernel(x, tables):
    tab128 = _tc_flatten(tables.transpose(0, 2, 1))
    tab = tab128.reshape(F * GP * 4, D)
    idx4 = x.reshape(NW, NJ, CH, F).transpose(0, 3, 1, 2)
    out_flat = _encode(tab, idx4)
    return out_flat.reshape(B, D)
